# Initial kernel scaffold; baseline (speedup 1.0000x reference)
import jax
import jax.numpy as jnp
from jax.experimental import pallas as pl


def kernel(nodes_feat, edges_feat, nodes_num_norm_sqrt, edges_num_norm_sqrt, W_emb_h, b_emb_h, W_emb_e, b_emb_e, layer_W, layer_b, bn_gamma_h, bn_beta_h, bn_gamma_e, bn_beta_e, W_r1, b_r1, W_r2, b_r2, W_r3, b_r3, edge_index):
    raise NotImplementedError("write your pallas kernel here")


# speedup vs baseline: 2.3049x; 2.3049x over previous
import functools

import jax
import jax.numpy as jnp
from jax import lax
from jax.experimental import pallas as pl
from jax.experimental.pallas import tpu as pltpu
from jax.experimental.pallas import tpu_sc as plsc

N = 10000
E = 160000
D_IN = 128
D_EDGE = 16
H = 128
L = 4
HH = H // 2


_CH = 80
_EPT = E // 16
_NCH = _EPT // _CH
_RPT = 624


def _make_sc(write_en: bool):
    mesh = plsc.VectorSubcoreMesh(core_axis_name="c", subcore_axis_name="s",
                                  num_cores=2, num_subcores=16)
    f32 = jnp.float32
    out_type = [
        jax.ShapeDtypeStruct((N, 2 * HH), f32),
        jax.ShapeDtypeStruct((N, 2 * HH), f32),
        jax.ShapeDtypeStruct((E, HH), f32),
        jax.ShapeDtypeStruct((E, HH), f32),
    ]

    @functools.partial(
        pl.kernel, mesh=mesh, out_type=out_type,
        scratch_types=[
            pltpu.VMEM((_CH,), jnp.int32),
            pltpu.VMEM((_CH,), jnp.int32),
            pltpu.VMEM((_CH, HH), f32),
            pltpu.VMEM((_CH, 2 * HH), f32),
            pltpu.VMEM((_CH, 2 * HH), f32),
            pltpu.VMEM((_CH, 2 * HH), f32),
            pltpu.VMEM_SHARED((N, 2 * HH), f32),
            pltpu.SemaphoreType.DMA,
        ],
        name="gcn_msg_sc" if write_en else "gcn_msg_sc_last",
    )
    def sc(ceL, ceR, dbL, dbR, etL, etR, src_h, dst_h,
           ndL, ndR, enL, enR,
           src_v, dst_v, ce_v, db_v, et_v, pay_v, acc, sem):
        c = lax.axis_index("c")
        s = lax.axis_index("s")
        tbase = s * _EPT
        rb = s * _RPT

        zv = jnp.zeros((16,), f32)

        def zrow(i, _):
            r = i // 8
            f = i % 8
            pay_v[r, pl.ds(f * 16, 16)] = zv
            return 0
        lax.fori_loop(0, _CH * 8, zrow, 0)
        for z in range(7):
            pltpu.sync_copy(pay_v, acc.at[pl.ds(rb + z * 80, 80)])
        pltpu.sync_copy(pay_v.at[pl.ds(0, 64)], acc.at[pl.ds(rb + 560, 64)])

        @pl.when(s == 15)
        def _():
            pltpu.sync_copy(pay_v.at[pl.ds(0, 16)], acc.at[pl.ds(N - 16, 16)])
        plsc.subcore_barrier()

        def chunk(j, _):
            base = tbase + j * _CH
            pltpu.sync_copy(src_h.at[pl.ds(base, _CH)], src_v)
            pltpu.sync_copy(dst_h.at[pl.ds(base, _CH)], dst_v)

            @pl.when(c == 0)
            def _():
                pltpu.sync_copy(ceL.at[pl.ds(base, _CH)], ce_v)
                g1 = pltpu.async_copy(dbL.at[src_v], db_v, sem)
                g2 = pltpu.async_copy(etL.at[dst_v], et_v, sem)
                g1.wait()
                g2.wait()

            @pl.when(c == 1)
            def _():
                pltpu.sync_copy(ceR.at[pl.ds(base, _CH)], ce_v)
                g1 = pltpu.async_copy(dbR.at[src_v], db_v, sem)
                g2 = pltpu.async_copy(etR.at[dst_v], et_v, sem)
                g1.wait()
                g2.wait()

            def edge(k, _):
                for f in range(HH // 16):
                    sl = pl.ds(f * 16, 16)
                    sh = pl.ds(HH + f * 16, 16)
                    x = ce_v[k, sl] + db_v[k, sl] + et_v[k, sl]
                    sig = 1.0 / (1.0 + jnp.exp(-x))
                    pay_v[k, sl] = sig * db_v[k, sh]
                    pay_v[k, sh] = sig
                    if write_en:
                        ce_v[k, sl] = x
                return 0
            lax.fori_loop(0, _CH, edge, 0)

            if write_en:
                @pl.when(c == 0)
                def _():
                    pltpu.sync_copy(ce_v, enL.at[pl.ds(base, _CH)])

                @pl.when(c == 1)
                def _():
                    pltpu.sync_copy(ce_v, enR.at[pl.ds(base, _CH)])

            pltpu.sync_copy(pay_v, acc.at[dst_v], add=True)
            return 0
        lax.fori_loop(0, _NCH, chunk, 0)

        plsc.subcore_barrier()

        @pl.when(c == 0)
        def _():
            pltpu.sync_copy(acc.at[pl.ds(rb, _RPT)], ndL.at[pl.ds(rb, _RPT)])

            @pl.when(s == 15)
            def _():
                pltpu.sync_copy(acc.at[pl.ds(N - 16, 16)],
                                ndL.at[pl.ds(N - 16, 16)])

        @pl.when(c == 1)
        def _():
            pltpu.sync_copy(acc.at[pl.ds(rb, _RPT)], ndR.at[pl.ds(rb, _RPT)])

            @pl.when(s == 15)
            def _():
                pltpu.sync_copy(acc.at[pl.ds(N - 16, 16)],
                                ndR.at[pl.ds(N - 16, 16)])

    return sc


_sc_cache = {}


def _get_sc(write_en: bool):
    if write_en not in _sc_cache:
        _sc_cache[write_en] = _make_sc(write_en)
    return _sc_cache[write_en]


_F32 = jnp.float32


def _node_front_body(x_ref, we_ref, be_ref, w5_ref, b5_ref,
                     h_ref, ah_ref, dbl_ref, dbr_ref, etl_ref, etr_ref):
    h = jnp.dot(x_ref[...], we_ref[...],
                preferred_element_type=_F32) + be_ref[...]
    h_ref[...] = h
    ah_ref[...] = jnp.dot(h, w5_ref[0], preferred_element_type=_F32) + b5_ref[0][None, :]
    bh = jnp.dot(h, w5_ref[1], preferred_element_type=_F32) + b5_ref[1][None, :]
    dh = jnp.dot(h, w5_ref[3], preferred_element_type=_F32) + b5_ref[3][None, :]
    eh = jnp.dot(h, w5_ref[4], preferred_element_type=_F32) + b5_ref[4][None, :]
    dbl_ref[...] = jnp.concatenate([dh[:, :HH], bh[:, :HH]], axis=1)
    dbr_ref[...] = jnp.concatenate([dh[:, HH:], bh[:, HH:]], axis=1)
    etl_ref[...] = eh
    etr_ref[...] = jnp.concatenate([eh[:, HH:], eh[:, :HH]], axis=1)


_node_front = pl.pallas_call(
    _node_front_body,
    out_shape=[
        jax.ShapeDtypeStruct((N, H), _F32),
        jax.ShapeDtypeStruct((N, H), _F32),
        jax.ShapeDtypeStruct((N, 2 * HH), _F32),
        jax.ShapeDtypeStruct((N, 2 * HH), _F32),
        jax.ShapeDtypeStruct((N, H), _F32),
        jax.ShapeDtypeStruct((N, H), _F32),
    ],
)


def _mm_tables_body(h_ref, w5_ref, b5_ref,
                    ah_ref, dbl_ref, dbr_ref, etl_ref, etr_ref):
    h = h_ref[...]
    ah_ref[...] = jnp.dot(h, w5_ref[0], preferred_element_type=_F32) + b5_ref[0][None, :]
    bh = jnp.dot(h, w5_ref[1], preferred_element_type=_F32) + b5_ref[1][None, :]
    dh = jnp.dot(h, w5_ref[3], preferred_element_type=_F32) + b5_ref[3][None, :]
    eh = jnp.dot(h, w5_ref[4], preferred_element_type=_F32) + b5_ref[4][None, :]
    dbl_ref[...] = jnp.concatenate([dh[:, :HH], bh[:, :HH]], axis=1)
    dbr_ref[...] = jnp.concatenate([dh[:, HH:], bh[:, HH:]], axis=1)
    etl_ref[...] = eh
    etr_ref[...] = jnp.concatenate([eh[:, HH:], eh[:, :HH]], axis=1)


_mm_tables = pl.pallas_call(
    _mm_tables_body,
    out_shape=[
        jax.ShapeDtypeStruct((N, H), _F32),
        jax.ShapeDtypeStruct((N, 2 * HH), _F32),
        jax.ShapeDtypeStruct((N, 2 * HH), _F32),
        jax.ShapeDtypeStruct((N, H), _F32),
        jax.ShapeDtypeStruct((N, H), _F32),
    ],
)


def _h_update(h, ah, ndl, ndr, nrm, gamma, beta):
    num = jnp.concatenate([ndl[:, :HH], ndr[:, :HH]], axis=1)
    den = jnp.concatenate([ndl[:, HH:], ndr[:, HH:]], axis=1)
    hn = (ah + num / (den + 1e-6)) * nrm
    mu = jnp.mean(hn, axis=0, keepdims=True)
    var = jnp.mean((hn - mu) ** 2, axis=0, keepdims=True)
    hb = gamma * (hn - mu) / jnp.sqrt(var + 1e-5) + beta
    return h + jnp.maximum(hb, 0.0)


def _hup_body(h_ref, ah_ref, ndl_ref, ndr_ref, nrm_ref, g_ref, b_ref, ho_ref):
    ho_ref[...] = _h_update(h_ref[...], ah_ref[...], ndl_ref[...], ndr_ref[...],
                            nrm_ref[...], g_ref[...], b_ref[...])


_hup = pl.pallas_call(
    _hup_body,
    out_shape=jax.ShapeDtypeStruct((N, H), _F32),
)


def _hup_read_body(h_ref, ah_ref, ndl_ref, ndr_ref, nrm_ref, g_ref, b_ref,
                   w1_ref, b1_ref, w2_ref, b2_ref, w3_ref, b3_ref, out_ref):
    h4 = _h_update(h_ref[...], ah_ref[...], ndl_ref[...], ndr_ref[...],
                   nrm_ref[...], g_ref[...], b_ref[...])
    hg = jnp.mean(h4, axis=0, keepdims=True)
    y = jnp.maximum(jnp.dot(hg, w1_ref[...], preferred_element_type=_F32)
                    + b1_ref[...], 0.0)
    y = jnp.maximum(jnp.dot(y, w2_ref[...], preferred_element_type=_F32)
                    + b2_ref[...], 0.0)
    out_ref[...] = jnp.dot(y, w3_ref[...], preferred_element_type=_F32) + b3_ref[...]


_hup_read = pl.pallas_call(
    _hup_read_body,
    out_shape=jax.ShapeDtypeStruct((1, 10), _F32),
)

_EB = 2000
_GE = E // _EB


def _edge0_body(ef_ref, we_ref, be_ref, wc_ref, bc_ref,
                e_ref, cel_ref, cer_ref):
    e = jnp.dot(ef_ref[...], we_ref[...],
                preferred_element_type=_F32) + be_ref[...]
    e_ref[...] = e
    ce = jnp.dot(e, wc_ref[...], preferred_element_type=_F32) + bc_ref[...]
    cel_ref[...] = ce[:, :HH]
    cer_ref[...] = ce[:, HH:]


_edge0 = pl.pallas_call(
    _edge0_body,
    grid=(_GE,),
    in_specs=[
        pl.BlockSpec((_EB, D_EDGE), lambda i: (i, 0)),
        pl.BlockSpec((D_EDGE, H), lambda i: (0, 0)),
        pl.BlockSpec((1, H), lambda i: (0, 0)),
        pl.BlockSpec((H, H), lambda i: (0, 0)),
        pl.BlockSpec((1, H), lambda i: (0, 0)),
    ],
    out_specs=[
        pl.BlockSpec((_EB, H), lambda i: (i, 0)),
        pl.BlockSpec((_EB, HH), lambda i: (i, 0)),
        pl.BlockSpec((_EB, HH), lambda i: (i, 0)),
    ],
    out_shape=[
        jax.ShapeDtypeStruct((E, H), _F32),
        jax.ShapeDtypeStruct((E, HH), _F32),
        jax.ShapeDtypeStruct((E, HH), _F32),
    ],
)


def _estats_body(enl_ref, enr_ref, nrm_ref, st_ref, acc_ref):
    i = pl.program_id(0)
    xn = jnp.concatenate([enl_ref[...], enr_ref[...]], axis=1) * nrm_ref[...]
    ssum = jnp.sum(xn, axis=0, keepdims=True)
    sq = jnp.sum(xn * xn, axis=0, keepdims=True)
    part = jnp.concatenate([ssum, sq], axis=0)

    @pl.when(i == 0)
    def _():
        acc_ref[...] = part

    @pl.when(i != 0)
    def _():
        acc_ref[...] = acc_ref[...] + part

    @pl.when(i == pl.num_programs(0) - 1)
    def _():
        st_ref[...] = acc_ref[...]


_estats = pl.pallas_call(
    _estats_body,
    grid=(_GE,),
    in_specs=[
        pl.BlockSpec((_EB, HH), lambda i: (i, 0)),
        pl.BlockSpec((_EB, HH), lambda i: (i, 0)),
        pl.BlockSpec((_EB, 1), lambda i: (i, 0)),
    ],
    out_specs=pl.BlockSpec((2, H), lambda i: (0, 0)),
    out_shape=jax.ShapeDtypeStruct((2, H), _F32),
    scratch_shapes=[pltpu.VMEM((2, H), _F32)],
)


def _edgeB_body(e_ref, enl_ref, enr_ref, nrm_ref, st_ref, g_ref, be_ref,
                wc_ref, bc_ref, eo_ref, cel_ref, cer_ref):
    st = st_ref[...]
    mu = st[0:1, :] * (1.0 / E)
    msq = st[1:2, :] * (1.0 / E)
    var = msq - mu * mu
    xn = jnp.concatenate([enl_ref[...], enr_ref[...]], axis=1) * nrm_ref[...]
    bn = g_ref[...] * (xn - mu) / jnp.sqrt(var + 1e-5) + be_ref[...]
    e = e_ref[...] + jnp.maximum(bn, 0.0)
    eo_ref[...] = e
    ce = jnp.dot(e, wc_ref[...], preferred_element_type=_F32) + bc_ref[...]
    cel_ref[...] = ce[:, :HH]
    cer_ref[...] = ce[:, HH:]


_edgeB = pl.pallas_call(
    _edgeB_body,
    grid=(_GE,),
    in_specs=[
        pl.BlockSpec((_EB, H), lambda i: (i, 0)),
        pl.BlockSpec((_EB, HH), lambda i: (i, 0)),
        pl.BlockSpec((_EB, HH), lambda i: (i, 0)),
        pl.BlockSpec((_EB, 1), lambda i: (i, 0)),
        pl.BlockSpec((2, H), lambda i: (0, 0)),
        pl.BlockSpec((1, H), lambda i: (0, 0)),
        pl.BlockSpec((1, H), lambda i: (0, 0)),
        pl.BlockSpec((H, H), lambda i: (0, 0)),
        pl.BlockSpec((1, H), lambda i: (0, 0)),
    ],
    out_specs=[
        pl.BlockSpec((_EB, H), lambda i: (i, 0)),
        pl.BlockSpec((_EB, HH), lambda i: (i, 0)),
        pl.BlockSpec((_EB, HH), lambda i: (i, 0)),
    ],
    out_shape=[
        jax.ShapeDtypeStruct((E, H), _F32),
        jax.ShapeDtypeStruct((E, HH), _F32),
        jax.ShapeDtypeStruct((E, HH), _F32),
    ],
)


def kernel(nodes_feat, edges_feat, nodes_num_norm_sqrt, edges_num_norm_sqrt,
           W_emb_h, b_emb_h, W_emb_e, b_emb_e, layer_W, layer_b,
           bn_gamma_h, bn_beta_h, bn_gamma_e, bn_beta_e,
           W_r1, b_r1, W_r2, b_r2, W_r3, b_r3, edge_index):
    src = edge_index[0]
    dst = edge_index[1]

    h, ah, dbl, dbr, etl, etr = _node_front(
        nodes_feat, W_emb_h, b_emb_h.reshape(1, H), layer_W[0], layer_b[0])
    e, cel, cer = _edge0(
        edges_feat, W_emb_e, b_emb_e.reshape(1, H),
        layer_W[0, 2], layer_b[0, 2].reshape(1, H))

    logits = None
    for l in range(L):
        last = l == L - 1
        sc = _get_sc(not last)
        ndl, ndr, enl, enr = sc(cel, cer, dbl, dbr, etl, etr, src, dst)
        gh = bn_gamma_h[l].reshape(1, H)
        bh = bn_beta_h[l].reshape(1, H)
        if last:
            logits = _hup_read(h, ah, ndl, ndr, nodes_num_norm_sqrt, gh, bh,
                               W_r1, b_r1.reshape(1, -1),
                               W_r2, b_r2.reshape(1, -1),
                               W_r3, b_r3.reshape(1, -1))
        else:
            h = _hup(h, ah, ndl, ndr, nodes_num_norm_sqrt, gh, bh)
            ah, dbl, dbr, etl, etr = _mm_tables(h, layer_W[l + 1],
                                                layer_b[l + 1])
            st = _estats(enl, enr, edges_num_norm_sqrt)
            e, cel, cer = _edgeB(e, enl, enr, edges_num_norm_sqrt, st,
                                 bn_gamma_e[l].reshape(1, H),
                                 bn_beta_e[l].reshape(1, H),
                                 layer_W[l + 1, 2],
                                 layer_b[l + 1, 2].reshape(1, H))
    return logits
